# baseline (speedup 1.0000x reference)
import jax
import jax.numpy as jnp
from jax import lax
from jax.experimental import pallas as pl
from jax.experimental.pallas import tpu as pltpu
from jax.experimental.pallas import tpu_sc as plsc

_W = 128
_NC = 2
_NS = 16
_NW = _NC * _NS
_NB = 4


def kernel(x, table):
    n = x.shape[0]
    d = table.shape[1]
    idx = x.reshape(n).astype(jnp.int32)

    n_full = n // _W
    rem = n - n_full * _W
    n_win = n_full + (1 if rem else 0)
    wins_per_w = -(-n_win // _NW)
    per_w = wins_per_w * _W
    last_cnt = n - per_w * (_NW - 1)
    supersteps = -(-wins_per_w // _NB)
    assert wins_per_w >= _NB and n_full - wins_per_w * (_NW - 1) >= _NB
    if rem:
        assert n_full // wins_per_w == _NW - 1 and rem % 8 == 0
    rem_rows = rem if rem else 8

    mesh = plsc.VectorSubcoreMesh(core_axis_name="c", subcore_axis_name="s")

    v = table.shape[0]
    scratch = (
        [pltpu.VMEM_SHARED((v, d), table.dtype)]
        + [pltpu.VMEM((per_w,), jnp.int32)]
        + [pltpu.VMEM((_W, d), table.dtype) for _ in range(_NB)]
        + [pltpu.VMEM((rem_rows, d), table.dtype)]
        + [pltpu.SemaphoreType.DMA for _ in range(2 * _NB)]
    )

    @pl.kernel(
        out_type=jax.ShapeDtypeStruct((n, d), table.dtype),
        mesh=mesh,
        scratch_types=scratch,
    )
    def gather_kernel(table_hbm, idx_hbm, out_hbm, table_sp, idx_v, *rest):
        bufs = rest[:_NB]
        rem_buf = rest[_NB]
        gsems = rest[_NB + 1:2 * _NB + 1]
        wsems = rest[2 * _NB + 1:]
        w = lax.axis_index("s") * _NC + lax.axis_index("c")
        full_mine = jnp.clip(n_full - wins_per_w * w, 0, wins_per_w)

        @pl.when(lax.axis_index("s") == 0)
        def _():
            pltpu.sync_copy(table_hbm, table_sp)

        plsc.subcore_barrier()

        @pl.when(w < _NW - 1)
        def _():
            pltpu.sync_copy(idx_hbm.at[pl.ds(w * per_w, per_w)], idx_v)

        @pl.when(w == _NW - 1)
        def _():
            pltpu.sync_copy(
                idx_hbm.at[pl.ds((_NW - 1) * per_w, last_cnt)],
                idx_v.at[pl.ds(0, last_cnt)],
            )

        def issue_gather(j, buf, sem):
            pltpu.async_copy(
                table_sp.at[idx_v.at[pl.ds(j * _W, _W)]], buf, sem)

        def wait_gather(buf, sem):
            pltpu.make_async_copy(out_hbm.at[pl.ds(0, _W)], buf, sem).wait()

        def issue_write(j, buf, sem):
            g = w * wins_per_w + j
            h = _W // 2
            pltpu.async_copy(buf.at[pl.ds(0, h)],
                             out_hbm.at[pl.ds(g * _W, h)], sem)
            pltpu.async_copy(buf.at[pl.ds(h, h)],
                             out_hbm.at[pl.ds(g * _W + h, h)], sem)

        def wait_write(buf, sem):
            pltpu.make_async_copy(buf, out_hbm.at[pl.ds(0, _W)], sem).wait()

        for i in range(_NB):
            issue_gather(i, bufs[i], gsems[i])
        for i in range(_NB):
            wait_gather(bufs[i], gsems[i])
            issue_write(i, bufs[i], wsems[i])

        @pl.loop(1, supersteps)
        def _(t):
            j_base = _NB * t
            for i in range(_NB):
                @pl.when(j_base + i < full_mine)
                def _(i=i):
                    wait_write(bufs[i], wsems[i])
                    issue_gather(j_base + i, bufs[i], gsems[i])
            for i in range(_NB):
                @pl.when(j_base + i < full_mine)
                def _(i=i):
                    wait_gather(bufs[i], gsems[i])
                    issue_write(j_base + i, bufs[i], wsems[i])

        for i in range(_NB):
            wait_write(bufs[i], wsems[i])

        if rem:
            @pl.when(w == _NW - 1)
            def _():
                lo = (n_full - wins_per_w * (_NW - 1)) * _W
                pltpu.async_copy(
                    table_sp.at[idx_v.at[pl.ds(lo, rem)]], rem_buf,
                    gsems[0]).wait()
                pltpu.sync_copy(
                    rem_buf, out_hbm.at[pl.ds(n_full * _W, rem)])

    return gather_kernel(table, idx)


# speedup vs baseline: 1.0158x; 1.0158x over previous
import jax
import jax.numpy as jnp
from jax import lax
from jax.experimental import pallas as pl
from jax.experimental.pallas import tpu as pltpu
from jax.experimental.pallas import tpu_sc as plsc

_W = 128
_NC = 2
_NS = 16
_NW = _NC * _NS
_NB = 4


def kernel(x, table):
    n = x.shape[0]
    d = table.shape[1]
    idx = x.reshape(n).astype(jnp.int32)

    n_full = n // _W
    rem = n - n_full * _W
    n_win = n_full + (1 if rem else 0)
    wins_per_w = -(-n_win // _NW)
    per_w = wins_per_w * _W
    last_cnt = n - per_w * (_NW - 1)
    supersteps = -(-wins_per_w // _NB)
    assert wins_per_w >= _NB and n_full - wins_per_w * (_NW - 1) >= _NB
    if rem:
        assert n_full // wins_per_w == _NW - 1 and rem % 8 == 0
    rem_rows = rem if rem else 8

    mesh = plsc.VectorSubcoreMesh(core_axis_name="c", subcore_axis_name="s")

    v = table.shape[0]
    scratch = (
        [pltpu.VMEM_SHARED((v, d), table.dtype)]
        + [pltpu.VMEM((per_w,), jnp.int32)]
        + [pltpu.VMEM((_W, d), table.dtype) for _ in range(_NB)]
        + [pltpu.VMEM((rem_rows, d), table.dtype)]
        + [pltpu.SemaphoreType.DMA for _ in range(2 * _NB)]
    )

    @pl.kernel(
        out_type=jax.ShapeDtypeStruct((n, d), table.dtype),
        mesh=mesh,
        scratch_types=scratch,
    )
    def gather_kernel(table_hbm, idx_hbm, out_hbm, table_sp, idx_v, *rest):
        bufs = rest[:_NB]
        rem_buf = rest[_NB]
        gsems = rest[_NB + 1:2 * _NB + 1]
        wsems = rest[2 * _NB + 1:]
        w = lax.axis_index("s") * _NC + lax.axis_index("c")
        full_mine = jnp.clip(n_full - wins_per_w * w, 0, wins_per_w)

        @pl.when(w < _NW - 1)
        def _():
            pltpu.async_copy(idx_hbm.at[pl.ds(w * per_w, per_w)], idx_v,
                             gsems[0])

        @pl.when(w == _NW - 1)
        def _():
            pltpu.async_copy(
                idx_hbm.at[pl.ds((_NW - 1) * per_w, last_cnt)],
                idx_v.at[pl.ds(0, last_cnt)],
                gsems[0],
            )

        @pl.when(lax.axis_index("s") == 0)
        def _():
            pltpu.sync_copy(table_hbm, table_sp)

        @pl.when(w < _NW - 1)
        def _():
            pltpu.make_async_copy(
                idx_hbm.at[pl.ds(0, per_w)], idx_v, gsems[0]).wait()

        @pl.when(w == _NW - 1)
        def _():
            pltpu.make_async_copy(
                idx_hbm.at[pl.ds(0, last_cnt)],
                idx_v.at[pl.ds(0, last_cnt)], gsems[0]).wait()

        plsc.subcore_barrier()

        def issue_gather(j, buf, sem):
            pltpu.async_copy(
                table_sp.at[idx_v.at[pl.ds(j * _W, _W)]], buf, sem)

        def wait_gather(buf, sem):
            pltpu.make_async_copy(out_hbm.at[pl.ds(0, _W)], buf, sem).wait()

        def issue_write(j, buf, sem):
            g = w * wins_per_w + j
            pltpu.async_copy(buf, out_hbm.at[pl.ds(g * _W, _W)], sem)

        def wait_write(buf, sem):
            pltpu.make_async_copy(buf, out_hbm.at[pl.ds(0, _W)], sem).wait()

        for i in range(_NB):
            issue_gather(i, bufs[i], gsems[i])
        for i in range(_NB):
            wait_gather(bufs[i], gsems[i])
            issue_write(i, bufs[i], wsems[i])

        @pl.loop(1, supersteps)
        def _(t):
            j_base = _NB * t
            for i in range(_NB):
                @pl.when(j_base + i < full_mine)
                def _(i=i):
                    wait_write(bufs[i], wsems[i])
                    issue_gather(j_base + i, bufs[i], gsems[i])
            for i in range(_NB):
                @pl.when(j_base + i < full_mine)
                def _(i=i):
                    wait_gather(bufs[i], gsems[i])
                    issue_write(j_base + i, bufs[i], wsems[i])

        for i in range(_NB):
            wait_write(bufs[i], wsems[i])

        if rem:
            @pl.when(w == _NW - 1)
            def _():
                lo = (n_full - wins_per_w * (_NW - 1)) * _W
                pltpu.async_copy(
                    table_sp.at[idx_v.at[pl.ds(lo, rem)]], rem_buf,
                    gsems[0]).wait()
                pltpu.sync_copy(
                    rem_buf, out_hbm.at[pl.ds(n_full * _W, rem)])

    return gather_kernel(table, idx)
